# baseline (speedup 1.0000x reference)
import functools

import jax
import jax.numpy as jnp
from jax import lax
from jax.experimental import pallas as pl
from jax.experimental.pallas import tpu as pltpu
from jax.experimental.pallas import tpu_sc as plsc

K = 8
B = 4
C = 128
N = 2048
TN = 256
TP = 128
M = B * N * K
EPS = 1e-5
_PREC = lax.Precision.HIGHEST
_DIST_PREC = lax.Precision.DEFAULT
_CPREC = lax.Precision.DEFAULT


def _knn_body(f_ref, xt_ref, idx_ref, *, boff):
    b = pl.program_id(0)
    x = f_ref[0]
    xt = xt_ref[0]

    xx_full = jnp.sum(x * x, axis=0, keepdims=True)
    xx_row = jnp.sum(xt * xt, axis=1, keepdims=True)
    inner = jnp.dot(xt, x, preferred_element_type=jnp.float32,
                    precision=_DIST_PREC)
    dist = 2.0 * inner - xx_row - xx_full

    col = lax.broadcasted_iota(jnp.int32, (TN, N), 1)
    idx_cols = []
    for _ in range(K):
        idx_j = jnp.argmax(dist, axis=1, keepdims=True)
        dist = jnp.where(col == idx_j, -jnp.inf, dist)
        idx_cols.append(idx_j)
    idx = jnp.concatenate(idx_cols, axis=1)
    idx_ref[0] = idx + (b + boff) * N


def _knn(feats, xt3, boff):
    nb = feats.shape[0]
    grid = (nb, N // TN)
    return pl.pallas_call(
        functools.partial(_knn_body, boff=boff),
        grid=grid,
        in_specs=[
            pl.BlockSpec((1, C, N), lambda b, t: (b, 0, 0)),
            pl.BlockSpec((1, TN, C), lambda b, t: (b, t, 0)),
        ],
        out_specs=pl.BlockSpec((1, TN, K), lambda b, t: (b, t, 0)),
        out_shape=jax.ShapeDtypeStruct((nb, N, K), jnp.int32),
    )(feats, xt3)


_NUM_SC = 2
_NUM_SUBCORES = 16
_NW = _NUM_SC * _NUM_SUBCORES
_CHUNK = 256


def _gather_body(table_hbm, idx_hbm, out_hbm, idx_v, rows_a, rows_b, gsem_a,
                 gsem_b, ssem_a, ssem_b, *, rows_per_w):
    wid = lax.axis_index("c") * _NUM_SUBCORES + lax.axis_index("s")
    base = wid * rows_per_w
    pltpu.sync_copy(idx_hbm.at[pl.ds(base, rows_per_w)], idx_v)

    bufs = (rows_a, rows_b)
    gsems = (gsem_a, gsem_b)
    ssems = (ssem_a, ssem_b)
    nchunk = rows_per_w // _CHUNK

    def gather(c):
        return (table_hbm.at[idx_v.at[pl.ds(c * _CHUNK, _CHUNK)]],
                bufs[c % 2], gsems[c % 2])

    def scatter(c):
        return (bufs[c % 2], out_hbm.at[pl.ds(base + c * _CHUNK, _CHUNK)],
                ssems[c % 2])

    pltpu.async_copy(*gather(0))
    for c in range(nchunk):
        if c + 1 < nchunk:
            if c >= 1:
                pltpu.make_async_copy(*scatter(c - 1)).wait()
            pltpu.async_copy(*gather(c + 1))
        pltpu.make_async_copy(*gather(c)).wait()
        pltpu.async_copy(*scatter(c))
    pltpu.make_async_copy(*scatter(nchunk - 2)).wait()
    pltpu.make_async_copy(*scatter(nchunk - 1)).wait()


def _sc_gather(table, fidx):
    m = fidx.shape[0]
    rows_per_w = m // _NW
    mesh = plsc.VectorSubcoreMesh(core_axis_name="c", subcore_axis_name="s")
    k = pl.kernel(
        functools.partial(_gather_body, rows_per_w=rows_per_w),
        out_type=jax.ShapeDtypeStruct((m, C), jnp.float32),
        mesh=mesh,
        scratch_types=[
            pltpu.VMEM((rows_per_w,), jnp.int32),
            pltpu.VMEM((_CHUNK, C), jnp.float32),
            pltpu.VMEM((_CHUNK, C), jnp.float32),
            pltpu.SemaphoreType.DMA,
            pltpu.SemaphoreType.DMA,
            pltpu.SemaphoreType.DMA,
            pltpu.SemaphoreType.DMA,
        ],
    )
    return k(table, fidx)


_NB = B * N // TP


def _cm_conv(f_ref, xt, a, iw_ref, w1b_ref, h1_scr, st_ref, i):
    ff = f_ref[...].reshape(TP * K, C)
    sg = jnp.dot(ff, iw_ref[...], preferred_element_type=jnp.float32,
                 precision=_PREC)
    sg3 = sg.reshape(TP, K, 1)
    mn = sg3[:, 0, :]
    for j in range(1, K):
        mn = jnp.minimum(mn, sg3[:, j, :])
    e3 = jnp.exp(mn[:, None, :] - sg3)
    den = e3[:, 0, :]
    for j in range(1, K):
        den = den + e3[:, j, :]
    w3 = e3 * (1.0 / den)[:, None, :]

    wf = w3.reshape(TP * K, 1)
    hb = jnp.dot(ff * wf, w1b_ref[...],
                 preferred_element_type=jnp.float32,
                 precision=_CPREC)
    h = hb.reshape(TP, K, 2 * C) + a[:, None, :]
    h1_scr[pl.ds(i * TP, TP)] = h.astype(jnp.bfloat16)

    hf = h.reshape(TP * K, 2 * C)
    s1 = jnp.sum(hf, axis=0, keepdims=True)
    s2 = jnp.sum(hf * hf, axis=0, keepdims=True)
    st_ref[...] += jnp.concatenate([s1, s2], axis=0)


def _cm_body(f0_ref, f1_ref, xt_ref, iw_ref, w1a_ref, w1b_ref, b1_ref,
             g_ref, be_ref, wc_ref, bc_ref, q_ref, st2_ref, h1_scr, st_scr):
    i = pl.program_id(0)

    @pl.when(i == 0)
    def _():
        st_scr[...] = jnp.zeros_like(st_scr)
        st2_ref[...] = jnp.zeros_like(st2_ref)

    @pl.when(i < _NB // 2)
    def _():
        a = jnp.dot(xt_ref[...], w1a_ref[...],
                    preferred_element_type=jnp.float32,
                    precision=_CPREC) + b1_ref[...]
        _cm_conv(f0_ref, xt_ref, a, iw_ref, w1b_ref, h1_scr, st_scr, i)

    @pl.when((i >= _NB // 2) & (i < _NB))
    def _():
        a = jnp.dot(xt_ref[...], w1a_ref[...],
                    preferred_element_type=jnp.float32,
                    precision=_CPREC) + b1_ref[...]
        _cm_conv(f1_ref, xt_ref, a, iw_ref, w1b_ref, h1_scr, st_scr, i)

    @pl.when(i >= _NB)
    def _():
        j = i - _NB
        st = st_scr[...]
        mean = st[0:1, :] * (1.0 / M)
        var = st[1:2, :] * (1.0 / M) - mean * mean
        inv = lax.rsqrt(var + EPS)
        scale = g_ref[...] * inv
        shift = be_ref[...] - mean * scale

        h = h1_scr[pl.ds(j * TP, TP)].astype(jnp.float32)
        h = jnp.maximum(h.reshape(TP * K, 2 * C) * scale + shift, 0.0)
        q = jnp.dot(h, wc_ref[...], preferred_element_type=jnp.float32,
                    precision=_CPREC) + bc_ref[...]
        s1 = jnp.sum(q, axis=0, keepdims=True)
        s2 = jnp.sum(q * q, axis=0, keepdims=True)
        st2_ref[...] += jnp.concatenate([s1, s2], axis=0)
        q_ref[...] = q.astype(jnp.bfloat16)


def _conv_mid(F0, F1, xt_rows, iw2, w1aT, w1bT, b1, g1, be1, wcT, bc):
    nh = _NB // 2
    grid = (2 * _NB,)
    return pl.pallas_call(
        _cm_body,
        grid=grid,
        in_specs=[
            pl.BlockSpec((TP, K, C),
                         lambda i: (jnp.clip(i, 0, nh - 1), 0, 0)),
            pl.BlockSpec((TP, K, C),
                         lambda i: (jnp.clip(i - nh, 0, nh - 1), 0, 0)),
            pl.BlockSpec((TP, C), lambda i: (jnp.clip(i, 0, _NB - 1), 0)),
            pl.BlockSpec((C, 1), lambda i: (0, 0)),
            pl.BlockSpec((C, 2 * C), lambda i: (0, 0)),
            pl.BlockSpec((C, 2 * C), lambda i: (0, 0)),
            pl.BlockSpec((1, 2 * C), lambda i: (0, 0)),
            pl.BlockSpec((1, 2 * C), lambda i: (0, 0)),
            pl.BlockSpec((1, 2 * C), lambda i: (0, 0)),
            pl.BlockSpec((2 * C, C), lambda i: (0, 0)),
            pl.BlockSpec((1, C), lambda i: (0, 0)),
        ],
        out_specs=[
            pl.BlockSpec((TP * K, C),
                         lambda i: (jnp.clip(i - _NB, 0, _NB - 1), 0)),
            pl.BlockSpec((2, C), lambda i: (0, 0)),
        ],
        out_shape=[
            jax.ShapeDtypeStruct((B * N * K, C), jnp.bfloat16),
            jax.ShapeDtypeStruct((2, C), jnp.float32),
        ],
        scratch_shapes=[
            pltpu.VMEM((_NB * TP, K, 2 * C), jnp.bfloat16),
            pltpu.VMEM((2, 2 * C), jnp.float32),
        ],
    )(F0, F1, xt_rows, iw2, w1aT, w1bT, b1, g1, be1, wcT, bc)


def _final_body(q_ref, st_ref, g_ref, be_ref, w2_ref, b2_ref, o_ref):
    st = st_ref[...]
    mean = st[0:1, :] * (1.0 / M)
    var = st[1:2, :] * (1.0 / M) - mean * mean
    inv = lax.rsqrt(var + EPS)
    scale = g_ref[...] * inv
    shift = be_ref[...] - mean * scale

    q = jnp.maximum(q_ref[...].astype(jnp.float32) * scale + shift, 0.0)
    z = jnp.dot(q, w2_ref[...], preferred_element_type=jnp.float32,
                precision=_CPREC) + b2_ref[...]
    z3 = z.reshape(TP, K, C)
    m = z3[:, 0, :]
    for j in range(1, K):
        m = jnp.maximum(m, z3[:, j, :])
    o_ref[...] = m


def _final(q, st2, g2, be2, w2T, b2):
    nrows = q.shape[0]
    grid = (nrows // (TP * K),)
    return pl.pallas_call(
        _final_body,
        grid=grid,
        in_specs=[
            pl.BlockSpec((TP * K, C), lambda i: (i, 0)),
            pl.BlockSpec((2, C), lambda i: (0, 0)),
            pl.BlockSpec((1, C), lambda i: (0, 0)),
            pl.BlockSpec((1, C), lambda i: (0, 0)),
            pl.BlockSpec((C, C), lambda i: (0, 0)),
            pl.BlockSpec((1, C), lambda i: (0, 0)),
        ],
        out_specs=pl.BlockSpec((TP, C), lambda i: (i, 0)),
        out_shape=jax.ShapeDtypeStruct((nrows // K, C), jnp.float32),
    )(q, st2, g2, be2, w2T, b2)


def kernel(feats, inner_w, inner_b, cf_w1, cf_b1, cf_g1, cf_be1, cf_w2,
           cf_b2, mlp_w1, mlp_b1, mlp_g, mlp_be, mlp_w2, mlp_b2):
    del inner_b
    xt3 = jnp.transpose(feats, (0, 2, 1))
    xt_rows = xt3.reshape(B * N, C)
    iw2 = inner_w.reshape(C, 1)

    w1aT = jnp.transpose(cf_w1[:, :C])
    w1bT = jnp.transpose(cf_w1[:, C:])
    b1 = cf_b1.reshape(1, 2 * C)
    wc = jnp.dot(mlp_w1, cf_w2, precision=_PREC)
    bc = (jnp.dot(mlp_w1, cf_b2, precision=_PREC) + mlp_b1).reshape(1, C)
    wcT = jnp.transpose(wc)

    hb = B // 2
    hpts = hb * N
    idxs = [_knn(feats[i * hb:(i + 1) * hb], xt3[i * hb:(i + 1) * hb],
                 i * hb) for i in range(2)]
    Fs = [_sc_gather(xt_rows, idx.reshape(hpts * K)) for idx in idxs]

    g1r, be1r = cf_g1.reshape(1, 2 * C), cf_be1.reshape(1, 2 * C)
    q, st2 = _conv_mid(Fs[0].reshape(hpts, K, C), Fs[1].reshape(hpts, K, C),
                       xt_rows, iw2, w1aT, w1bT, b1, g1r, be1r, wcT, bc)

    g2r, be2r = mlp_g.reshape(1, C), mlp_be.reshape(1, C)
    w2T, b2r = jnp.transpose(mlp_w2), mlp_b2.reshape(1, C)
    rows = _final(q, st2, g2r, be2r, w2T, b2r)
    out = jnp.transpose(rows.reshape(B, N, C), (0, 2, 1))[:, :, :, None]
    return out
